# Initial kernel scaffold; baseline (speedup 1.0000x reference)
import jax
import jax.numpy as jnp
from jax.experimental import pallas as pl


def kernel(obs, node_obs, adj, agent_id, rnn_states, ssm_state_re, ssm_state_im, disturbances, masks, W1, W2, Wd, Wd2, Wm1, bm1, Wm2, bm2, Wa, ba, log_std, lam_re, lam_im, B_re, B_im, C_re, C_im, D, Wsm, Wso, bso):
    raise NotImplementedError("write your pallas kernel here")


# speedup vs baseline: 1.7333x; 1.7333x over previous
import functools

import jax
import jax.numpy as jnp
import numpy as np
from jax import lax
from jax.experimental import pallas as pl
from jax.experimental.pallas import tpu as pltpu
from jax.experimental.pallas import tpu_sc as plsc

_N = 32
_F = 16
_OBS = 16
_HID = 64
_ACT = 2
_KP = 1.0
_MMAX = 1.0
_BB = 512


def _sc_gather(adj_flat, nobs_flat, agent_id_flat):
    B = agent_id_flat.shape[0]
    info = plsc.get_sparse_core_info()
    nc, ns, L = info.num_cores, info.num_subcores, info.num_lanes
    nw = nc * ns
    bpw = B // nw
    mesh = plsc.VectorSubcoreMesh(core_axis_name="c", subcore_axis_name="s")

    @functools.partial(
        pl.kernel,
        mesh=mesh,
        compiler_params=pltpu.CompilerParams(use_tc_tiling_on_sc=False),
        out_type=[
            jax.ShapeDtypeStruct((B, _N), jnp.float32),
            jax.ShapeDtypeStruct((B, _F), jnp.float32),
        ],
        scratch_types=[
            pltpu.VMEM((bpw,), jnp.int32),
            pltpu.VMEM((bpw,), jnp.int32),
            pltpu.VMEM((bpw, _N), jnp.float32),
            pltpu.VMEM((bpw, _F), jnp.float32),
            pltpu.SemaphoreType.DMA,
            pltpu.SemaphoreType.DMA,
        ],
    )
    def k(adj_hbm, nobs_hbm, aid_hbm, adjrow_out, nobsag_out,
          aid_v, idx_v, adjrow_v, nobsag_v, sem_a, sem_n):
        wid = lax.axis_index("s") * nc + lax.axis_index("c")
        base = wid * bpw
        pltpu.sync_copy(aid_hbm.at[pl.ds(base, bpw)], aid_v)
        lane = lax.iota(jnp.int32, L) * _N
        for i in range(bpw // L):
            ids = aid_v[pl.ds(i * L, L)]
            idx_v[pl.ds(i * L, L)] = ids + lane + (base + i * L) * _N
        ca = pltpu.async_copy(adj_hbm.at[idx_v], adjrow_v, sem_a)
        cn = pltpu.async_copy(nobs_hbm.at[idx_v], nobsag_v, sem_n)
        ca.wait()
        cn.wait()
        pltpu.sync_copy(adjrow_v, adjrow_out.at[pl.ds(base, bpw)])
        pltpu.sync_copy(nobsag_v, nobsag_out.at[pl.ds(base, bpw)])

    return k(adj_flat, nobs_flat, agent_id_flat)


def _tc_body(obs_r, nobs_r, arow_r, nobsag_r, dist_r, sre_r, sim_r, msk_r,
             W1_r, W2_r, Wd_r, Wd2_r, Wm1_r, bm1_r, Wm2_r, bm2_r, Wa_r, ba_r,
             lstd_r, lre_r, lim_r, Bre_r, Bim_r, Cre_r, Cim_r, D_r, Wsm_r,
             Wso_r, bso_r, act_out, alp_out, nsre_out, nsim_out):
    f32 = jnp.float32

    def dot(a, b):
        return lax.dot_general(a, b, (((1,), (0,)), ((), ())),
                               preferred_element_type=f32)

    def bf(a):
        return a.astype(jnp.bfloat16).astype(f32)

    nobs = nobs_r[...]
    W1 = W1_r[...]
    h = jax.nn.relu(dot(nobs.reshape(_BB * _N, _F), W1))
    h3 = h.reshape(_BB, _N, _HID)
    arow = bf(arow_r[...])
    agg = jnp.sum(bf(h3) * arow[:, :, None], axis=1)
    h_ag = jax.nn.relu(dot(nobsag_r[...], W1))
    W2 = W2_r[...]
    nbd = jax.nn.relu(dot(h_ag, W2[:_HID]) + dot(agg, W2[_HID:]))
    obs = obs_r[...]
    Wm1 = Wm1_r[...]
    x = jax.nn.relu(dot(obs, Wm1[:_OBS]) + dot(nbd, Wm1[_OBS:]) + bm1_r[...])
    actor = jax.nn.relu(dot(x, Wm2_r[...]) + bm2_r[...])
    mean = dot(actor, Wa_r[...]) + ba_r[...]
    u_gnn = jnp.tanh(mean)

    dn = jax.nn.relu(dot(dist_r[...], Wd_r[...]))
    rs = jnp.sum(arow, axis=1, keepdims=True)
    Wd2 = Wd2_r[...]
    mag_g = jax.nn.relu(dot(dn, Wd2[:_HID]) + dot(rs * bf(dn), Wd2[_HID:]))

    reset = msk_r[...] == 0.0
    s_re = jnp.where(reset, 0.0, sre_r[...])
    s_im = jnp.where(reset, 0.0, sim_r[...])
    lre = lre_r[...]
    lim = lim_r[...]
    ns_re = lre * s_re - lim * s_im + dot(mag_g, Bre_r[...])
    ns_im = lre * s_im + lim * s_re + dot(mag_g, Bim_r[...])
    y_lru = dot(ns_re, Cre_r[...]) - dot(ns_im, Cim_r[...]) + dot(mag_g, D_r[...])
    ssm_raw = dot(jax.nn.relu(dot(y_lru, Wsm_r[...])), Wso_r[...]) + bso_r[...]
    magnitude = jnp.clip(jax.nn.relu(ssm_raw), 1e-6, _MMAX)

    actions = obs[:, 4:6] * _KP + magnitude * u_gnn
    lp = jnp.sum(-lstd_r[...] - 0.5 * np.float32(np.log(2.0 * np.pi)))
    ljt = jnp.sum(jnp.log(1.0 - u_gnn * u_gnn + 1e-8), axis=-1, keepdims=True)
    ljm = jnp.log(magnitude + 1e-8) * float(_ACT)
    act_out[...] = actions
    alp_out[...] = lp - ljm - ljt
    nsre_out[...] = ns_re
    nsim_out[...] = ns_im


def _tc_call(obs, node_obs, adj_rows, nobs_agent, dist, s_re, s_im, masks,
             W1, W2, Wd, Wd2, Wm1, bm1, Wm2, bm2, Wa, ba, lstd, lre, lim,
             Bre, Bim, Cre, Cim, D, Wsm, Wso, bso):
    B = obs.shape[0]
    grid = (B // _BB,)

    def bspec(shape):
        return pl.BlockSpec((_BB,) + shape[1:],
                            lambda i: (i,) + (0,) * (len(shape) - 1))

    def wspec(shape):
        return pl.BlockSpec(shape, lambda i: (0,) * len(shape))

    batch_args = [obs, node_obs, adj_rows, nobs_agent, dist, s_re, s_im, masks]
    weight_args = [W1, W2, Wd, Wd2, Wm1, bm1, Wm2, bm2, Wa, ba, lstd, lre,
                   lim, Bre, Bim, Cre, Cim, D, Wsm, Wso, bso]
    in_specs = [bspec(a.shape) for a in batch_args] + \
               [wspec(a.shape) for a in weight_args]
    out_shapes = [
        jax.ShapeDtypeStruct((B, _ACT), jnp.float32),
        jax.ShapeDtypeStruct((B, 1), jnp.float32),
        jax.ShapeDtypeStruct((B, _HID), jnp.float32),
        jax.ShapeDtypeStruct((B, _HID), jnp.float32),
    ]
    out_specs = [bspec(s.shape) for s in out_shapes]
    return pl.pallas_call(
        _tc_body,
        grid=grid,
        in_specs=in_specs,
        out_specs=out_specs,
        out_shape=out_shapes,
    )(*batch_args, *weight_args)


def kernel(obs, node_obs, adj, agent_id, rnn_states, ssm_state_re,
           ssm_state_im, disturbances, masks, W1, W2, Wd, Wd2, Wm1, bm1, Wm2,
           bm2, Wa, ba, log_std, lam_re, lam_im, B_re, B_im, C_re, C_im, D,
           Wsm, Wso, bso):
    B = obs.shape[0]
    adj_rows, nobs_agent = _sc_gather(
        adj.reshape(B * _N, _N),
        node_obs.reshape(B * _N, _F),
        agent_id.reshape(B).astype(jnp.int32),
    )
    actions, alp, ns_re, ns_im = _tc_call(
        obs, node_obs, adj_rows, nobs_agent, disturbances,
        ssm_state_re, ssm_state_im, masks,
        W1, W2, Wd, Wd2, Wm1, bm1.reshape(1, _HID), Wm2,
        bm2.reshape(1, _HID), Wa,
        ba.reshape(1, _ACT), log_std.reshape(1, _ACT),
        lam_re.reshape(1, _HID), lam_im.reshape(1, _HID),
        B_re, B_im, C_re, C_im, D, Wsm, Wso, bso.reshape(1, 1),
    )
    return (actions, alp, rnn_states, ns_re, ns_im)
